# baseline (speedup 1.0000x reference)
import functools

import numpy as np
import jax
import jax.numpy as jnp
from jax import lax
from jax.experimental import pallas as pl
from jax.experimental.pallas import tpu as pltpu
from jax.experimental.pallas import tpu_sc as plsc

_RATE = 0.25
_CH = 112
_L = 16


@functools.lru_cache(maxsize=None)
def _sample_rows(B, N, nsample):
    with jax.ensure_compile_time_eval():
        skey = jax.random.key(42)
        idx = np.stack(
            [np.asarray(jax.random.permutation(jax.random.fold_in(skey, b), N)[:nsample])
             for b in range(B)],
            axis=0,
        ).astype(np.int64)
    return idx


@functools.lru_cache(maxsize=None)
def _index_tables(B, N, nsample, ns_pad, tot_pad):
    idx = _sample_rows(B, N, nsample)
    gidx = np.zeros((tot_pad,), np.int32)
    gidx[: B * nsample] = (idx + (np.arange(B, dtype=np.int64) * N)[:, None]).reshape(-1)
    lidx = np.zeros((B, 1, ns_pad), np.int32)
    lidx[:, 0, :nsample] = idx
    return gidx, lidx


def kernel(xyz, feature):
    B, N, DX = xyz.shape
    _, _, DF = feature.shape
    nsample = int(_RATE * N)
    tot = B * nsample

    mesh = plsc.VectorSubcoreMesh(core_axis_name="c", subcore_axis_name="s")
    nc, ns = mesh.num_cores, mesh.num_subcores
    nw = nc * ns

    pw = -(-tot // (nw * _CH)) * _CH
    nch = pw // _CH
    tot_pad = pw * nw

    npl = DX * B
    ns_pad = -(-nsample // _L) * _L
    nstep = ns_pad // _L

    gidx_np, lidx_np = _index_tables(B, N, nsample, ns_pad, tot_pad)
    gidx = jnp.asarray(gidx_np).reshape(nw, nch, _CH)
    lidx = jnp.asarray(lidx_np)

    feat_flat = feature.reshape(B * N, DF)
    eye = jnp.eye(DX, dtype=xyz.dtype)
    planes = lax.dot_general(
        eye, xyz, (((1,), (2,)), ((), ())),
        precision=lax.Precision.HIGHEST).reshape(npl, 1, N)

    @functools.partial(
        pl.kernel,
        out_type=(
            jax.ShapeDtypeStruct((npl, 1, ns_pad), xyz.dtype),
            jax.ShapeDtypeStruct((tot_pad, DF), feature.dtype),
        ),
        mesh=mesh,
        compiler_params=pltpu.CompilerParams(needs_layout_passes=False),
        scratch_types=[
            pltpu.VMEM((nch, _CH), jnp.int32),
            pltpu.VMEM((_CH, DF), jnp.float32),
            pltpu.VMEM((_CH, DF), jnp.float32),
            pltpu.VMEM((_CH, DF), jnp.float32),
            pltpu.VMEM((1, N), jnp.float32),
            pltpu.VMEM((1, ns_pad), jnp.int32),
            pltpu.VMEM((1, ns_pad), jnp.float32),
            pltpu.SemaphoreType.DMA,
            pltpu.SemaphoreType.DMA,
            pltpu.SemaphoreType.DMA,
            pltpu.SemaphoreType.DMA,
            pltpu.SemaphoreType.DMA,
            pltpu.SemaphoreType.DMA,
        ],
    )
    def gather_rows(planes_hbm, feat_hbm, gidx_hbm, lidx_hbm, xout_hbm, fout_hbm,
                    idx_v, fbuf0, fbuf1, fbuf2, plane_v, lidx_v, xres_v,
                    sin0, sin1, sin2, sout0, sout1, sout2):
        wid = lax.axis_index("s") * nc + lax.axis_index("c")
        fbufs = (fbuf0, fbuf1, fbuf2)
        sins = (sin0, sin1, sin2)
        souts = (sout0, sout1, sout2)
        base = wid * nch * _CH

        pltpu.sync_copy(gidx_hbm.at[wid], idx_v)
        cin = {c: pltpu.async_copy(feat_hbm.at[idx_v.at[c]], fbufs[c], sins[c])
               for c in range(min(2, nch))}
        cout = {}

        @pl.when(wid < npl)
        def _xyz():
            b = lax.rem(wid, B)
            pltpu.sync_copy(planes_hbm.at[wid], plane_v)
            pltpu.sync_copy(lidx_hbm.at[b], lidx_v)
            zero16 = jnp.zeros((_L,), jnp.int32)

            def step(j, carry):
                ids = lidx_v[0, pl.ds(j * _L, _L)]
                xres_v[0, pl.ds(j * _L, _L)] = plsc.load_gather(plane_v, [zero16, ids])
                return carry

            lax.fori_loop(0, nstep, step, 0)
            pltpu.sync_copy(xres_v, xout_hbm.at[wid])

        for c in range(nch):
            cin[c].wait()
            if c + 2 < nch:
                if c >= 1:
                    cout[c - 1].wait()
                cin[c + 2] = pltpu.async_copy(
                    feat_hbm.at[idx_v.at[c + 2]], fbufs[(c + 2) % 3], sins[(c + 2) % 3])
            cout[c] = pltpu.async_copy(
                fbufs[c % 3], fout_hbm.at[pl.ds(base + c * _CH, _CH)], souts[c % 3])
        for c in range(max(0, nch - 3), nch):
            cout[c].wait()

    xout, fout = gather_rows(planes, feat_flat, gidx, lidx)
    sampled_xyz = xout.reshape(DX, B, ns_pad)[:, :, :nsample].transpose(1, 2, 0)
    sampled_feature = fout[:tot].reshape(B, nsample, DF)
    return sampled_xyz, sampled_feature


# speedup vs baseline: 1.0000x; 1.0000x over previous
import functools

import numpy as np
import jax
import jax.numpy as jnp
from jax import lax
from jax.experimental import pallas as pl
from jax.experimental.pallas import tpu as pltpu
from jax.experimental.pallas import tpu_sc as plsc

_RATE = 0.25
_CH = 112
_L = 16


@functools.lru_cache(maxsize=None)
def _sample_rows(B, N, nsample):
    with jax.ensure_compile_time_eval():
        skey = jax.random.key(42)
        idx = np.stack(
            [np.asarray(jax.random.permutation(jax.random.fold_in(skey, b), N)[:nsample])
             for b in range(B)],
            axis=0,
        ).astype(np.int64)
    return idx


@functools.lru_cache(maxsize=None)
def _index_tables(B, N, nsample, ns_pad, tot_pad):
    idx = _sample_rows(B, N, nsample)
    gidx = np.zeros((tot_pad,), np.int32)
    gidx[: B * nsample] = (idx + (np.arange(B, dtype=np.int64) * N)[:, None]).reshape(-1)
    lidx = np.zeros((B, 1, ns_pad), np.int32)
    lidx[:, 0, :nsample] = idx
    return gidx, lidx


def kernel(xyz, feature):
    B, N, DX = xyz.shape
    _, _, DF = feature.shape
    nsample = int(_RATE * N)
    tot = B * nsample

    mesh = plsc.VectorSubcoreMesh(core_axis_name="c", subcore_axis_name="s")
    nc, ns = mesh.num_cores, mesh.num_subcores
    nw = nc * ns

    pw = -(-tot // (nw * _CH)) * _CH
    nch = pw // _CH
    tot_pad = pw * nw

    npl = DX * B
    ns_pad = -(-nsample // _L) * _L
    nstep = ns_pad // _L

    gidx_np, lidx_np = _index_tables(B, N, nsample, ns_pad, tot_pad)
    gidx = jnp.asarray(gidx_np).reshape(nw, nch, _CH)
    lidx = jnp.asarray(lidx_np)

    feat_flat = feature.reshape(B * N, DF)
    eye = jnp.eye(DX, dtype=xyz.dtype)
    planes = lax.dot_general(
        eye, xyz, (((1,), (2,)), ((), ())),
        precision=lax.Precision.HIGHEST).reshape(npl, 1, N)

    @functools.partial(
        pl.kernel,
        out_type=(
            jax.ShapeDtypeStruct((npl, 1, ns_pad), xyz.dtype),
            jax.ShapeDtypeStruct((tot_pad, DF), feature.dtype),
        ),
        mesh=mesh,
        compiler_params=pltpu.CompilerParams(needs_layout_passes=False),
        scratch_types=[
            pltpu.VMEM((nch, _CH), jnp.int32),
            pltpu.VMEM((_CH, DF), jnp.float32),
            pltpu.VMEM((_CH, DF), jnp.float32),
            pltpu.VMEM((_CH, DF), jnp.float32),
            pltpu.VMEM((1, N), jnp.float32),
            pltpu.VMEM((1, ns_pad), jnp.int32),
            pltpu.VMEM((1, ns_pad), jnp.float32),
            pltpu.SemaphoreType.DMA,
            pltpu.SemaphoreType.DMA,
            pltpu.SemaphoreType.DMA,
            pltpu.SemaphoreType.DMA,
            pltpu.SemaphoreType.DMA,
            pltpu.SemaphoreType.DMA,
        ],
    )
    def gather_rows(planes_hbm, feat_hbm, gidx_hbm, lidx_hbm, xout_hbm, fout_hbm,
                    idx_v, fbuf0, fbuf1, fbuf2, plane_v, lidx_v, xres_v,
                    sin0, sin1, sin2, sout0, sout1, sout2):
        wid = lax.axis_index("s") * nc + lax.axis_index("c")
        fbufs = (fbuf0, fbuf1, fbuf2)
        sins = (sin0, sin1, sin2)
        souts = (sout0, sout1, sout2)
        base = wid * nch * _CH

        pltpu.sync_copy(gidx_hbm.at[wid], idx_v)
        cin = {c: pltpu.async_copy(feat_hbm.at[idx_v.at[c]], fbufs[c], sins[c])
               for c in range(min(2, nch))}
        cout = {}

        @pl.when(wid < npl)
        def _xyz():
            b = lax.rem(wid, B)
            pltpu.sync_copy(planes_hbm.at[wid], plane_v)
            pltpu.sync_copy(lidx_hbm.at[b], lidx_v)
            zero16 = jnp.zeros((_L,), jnp.int32)

            def step(j, carry):
                ids = lidx_v[0, pl.ds(j * _L, _L)]
                xres_v[0, pl.ds(j * _L, _L)] = plsc.load_gather(plane_v, [zero16, ids])
                return carry

            lax.fori_loop(0, nstep, step, 0)
            pltpu.sync_copy(xres_v, xout_hbm.at[wid])

        for c in range(nch):
            cin[c].wait()
            if c + 2 < nch:
                if c >= 1:
                    cout[c - 1].wait()
                cin[c + 2] = pltpu.async_copy(
                    feat_hbm.at[idx_v.at[c + 2]], fbufs[(c + 2) % 3], sins[(c + 2) % 3])
            cout[c] = pltpu.async_copy(
                fbufs[c % 3], fout_hbm.at[pl.ds(base + c * _CH, _CH)], souts[c % 3])
        for c in range(max(0, nch - 3), nch):
            cout[c].wait()

    xout, fout = gather_rows(planes, feat_flat, gidx, lidx)
    xs = xout.reshape(DX, B, ns_pad)
    xyz_t = lax.dot_general(
        xs, eye, (((0,), (0,)), ((), ())), precision=lax.Precision.HIGHEST)
    sampled_xyz = xyz_t[:, :nsample, :]
    sampled_feature = fout[:tot].reshape(B, nsample, DF)
    return sampled_xyz, sampled_feature


# speedup vs baseline: 1.7192x; 1.7191x over previous
import functools

import numpy as np
import jax
import jax.numpy as jnp
from jax import lax
from jax.experimental import pallas as pl
from jax.experimental.pallas import tpu as pltpu
from jax.experimental.pallas import tpu_sc as plsc

_RATE = 0.25
_CH = 112
_L = 16


@functools.lru_cache(maxsize=None)
def _sample_rows(B, N, nsample):
    with jax.ensure_compile_time_eval():
        skey = jax.random.key(42)
        idx = np.stack(
            [np.asarray(jax.random.permutation(jax.random.fold_in(skey, b), N)[:nsample])
             for b in range(B)],
            axis=0,
        ).astype(np.int64)
    return idx


@functools.lru_cache(maxsize=None)
def _index_tables(B, N, nsample, ns_pad, nsb, nchb):
    idx = _sample_rows(B, N, nsample)
    gidx = np.zeros((B, nchb, _CH), np.int64)
    for b in range(B):
        for k in range(nchb):
            s = min(k * _CH, nsb - _CH)
            pos = np.minimum(np.arange(s, s + _CH), nsample - 1)
            gidx[b, k, :] = idx[b, pos] + b * N
    lidx = np.zeros((B, 1, ns_pad), np.int32)
    lidx[:, 0, :nsample] = idx
    return gidx.astype(np.int32), lidx


def kernel(xyz, feature):
    B, N, DX = xyz.shape
    _, _, DF = feature.shape
    nsample = int(_RATE * N)
    tot = B * nsample

    mesh = plsc.VectorSubcoreMesh(core_axis_name="c", subcore_axis_name="s")
    nc, ns = mesh.num_cores, mesh.num_subcores
    nw = nc * ns

    nsb = -(-nsample // 8) * 8
    nchb = -(-nsb // _CH)
    nch = B * nchb // nw
    assert nch * nw == B * nchb

    npl = DX * B
    ns_pad = -(-nsample // _L) * _L
    nstep = ns_pad // _L

    gidx_np, lidx_np = _index_tables(B, N, nsample, ns_pad, nsb, nchb)
    gidx = jnp.asarray(gidx_np).reshape(nw, nch, _CH)
    lidx = jnp.asarray(lidx_np)

    feat_flat = feature.reshape(B * N, DF)
    eye = jnp.eye(DX, dtype=xyz.dtype)
    planes = lax.dot_general(
        eye, xyz, (((1,), (2,)), ((), ())),
        precision=lax.Precision.HIGHEST).reshape(npl, 1, N)

    @functools.partial(
        pl.kernel,
        out_type=(
            jax.ShapeDtypeStruct((npl, 1, ns_pad), xyz.dtype),
            jax.ShapeDtypeStruct((B, nsb, DF), feature.dtype),
        ),
        mesh=mesh,
        compiler_params=pltpu.CompilerParams(needs_layout_passes=False),
        scratch_types=[
            pltpu.VMEM((nch, _CH), jnp.int32),
            pltpu.VMEM((_CH, DF), jnp.float32),
            pltpu.VMEM((_CH, DF), jnp.float32),
            pltpu.VMEM((_CH, DF), jnp.float32),
            pltpu.VMEM((1, N), jnp.float32),
            pltpu.VMEM((1, ns_pad), jnp.int32),
            pltpu.VMEM((1, ns_pad), jnp.float32),
            pltpu.SemaphoreType.DMA,
            pltpu.SemaphoreType.DMA,
            pltpu.SemaphoreType.DMA,
            pltpu.SemaphoreType.DMA,
            pltpu.SemaphoreType.DMA,
            pltpu.SemaphoreType.DMA,
        ],
    )
    def gather_rows(planes_hbm, feat_hbm, gidx_hbm, lidx_hbm, xout_hbm, fout_hbm,
                    idx_v, fbuf0, fbuf1, fbuf2, plane_v, lidx_v, xres_v,
                    sin0, sin1, sin2, sout0, sout1, sout2):
        wid = lax.axis_index("s") * nc + lax.axis_index("c")
        fbufs = (fbuf0, fbuf1, fbuf2)
        sins = (sin0, sin1, sin2)
        souts = (sout0, sout1, sout2)

        def out_slot(c):
            j = wid * nch + c
            bb = lax.div(j, nchb)
            s = lax.min(lax.rem(j, nchb) * _CH, nsb - _CH)
            return fout_hbm.at[bb, pl.ds(s, _CH)]

        pltpu.sync_copy(gidx_hbm.at[wid], idx_v)
        cin = {c: pltpu.async_copy(feat_hbm.at[idx_v.at[c]], fbufs[c], sins[c])
               for c in range(min(2, nch))}
        cout = {}

        @pl.when(wid < npl)
        def _xyz():
            b = lax.rem(wid, B)
            pltpu.sync_copy(planes_hbm.at[wid], plane_v)
            pltpu.sync_copy(lidx_hbm.at[b], lidx_v)
            zero16 = jnp.zeros((_L,), jnp.int32)

            def step(j, carry):
                ids = lidx_v[0, pl.ds(j * _L, _L)]
                xres_v[0, pl.ds(j * _L, _L)] = plsc.load_gather(plane_v, [zero16, ids])
                return carry

            lax.fori_loop(0, nstep, step, 0)
            pltpu.sync_copy(xres_v, xout_hbm.at[wid])

        for c in range(nch):
            cin[c].wait()
            if c + 2 < nch:
                if c >= 1:
                    cout[c - 1].wait()
                cin[c + 2] = pltpu.async_copy(
                    feat_hbm.at[idx_v.at[c + 2]], fbufs[(c + 2) % 3], sins[(c + 2) % 3])
            cout[c] = pltpu.async_copy(fbufs[c % 3], out_slot(c), souts[c % 3])
        for c in range(max(0, nch - 3), nch):
            cout[c].wait()

    xout, fout = gather_rows(planes, feat_flat, gidx, lidx)
    xs = xout.reshape(DX, B, ns_pad)
    xyz_t = lax.dot_general(
        xs, eye, (((0,), (0,)), ((), ())), precision=lax.Precision.HIGHEST)
    sampled_xyz = xyz_t[:, :nsample, :]
    sampled_feature = fout[:, :nsample, :]
    return sampled_xyz, sampled_feature
